# baseline (speedup 1.0000x reference)
import jax
import jax.numpy as jnp
from jax import lax
from jax.experimental import pallas as pl
from jax.experimental.pallas import tpu as pltpu
from jax.experimental.pallas import tpu_sc as plsc

_MAX_VAL = 10
_EMB = 128
_BATCH = 16384
_NROWS = 2 * _MAX_VAL + 1

_NC = 2
_NS = 16
_NW = _NC * _NS
_BPW = _BATCH // _NW
_CH = 4
_CB = _BPW // _CH
_L = 16


def _body(values_hbm, table_hbm, out_hbm, vals_v, idx_v, table_sh, rows_v,
          gsems, wsem, tsem):
    wid = lax.axis_index("s") * _NC + lax.axis_index("c")
    base = wid * _BPW
    tcopy = pltpu.async_copy(table_hbm, table_sh, tsem)
    pltpu.sync_copy(values_hbm.at[pl.ds(base, _BPW)], vals_v)
    for j in range(_CH):
        for k in range(_CB // _L):
            idx_v[j, pl.ds(k * _L, _L)] = (
                vals_v[pl.ds(j * _CB + k * _L, _L)] + _MAX_VAL
            )
    tcopy.wait()
    gathers = [
        pltpu.async_copy(
            table_sh.at[idx_v.at[j]], rows_v.at[pl.ds(j * _CB, _CB)], gsems[j]
        )
        for j in range(_CH)
    ]
    writes = []
    for j in range(_CH):
        gathers[j].wait()
        writes.append(
            pltpu.async_copy(
                rows_v.at[pl.ds(j * _CB, _CB)],
                out_hbm.at[pl.ds(base + j * _CB, _CB)],
                wsem,
            )
        )
    for w in writes:
        w.wait()


@jax.jit
def kernel(values, emb_table):
    run = pl.kernel(
        _body,
        mesh=plsc.VectorSubcoreMesh(core_axis_name="c", subcore_axis_name="s"),
        compiler_params=pltpu.CompilerParams(
            needs_layout_passes=False,
            disable_bounds_checks=True,
            disable_semaphore_checks=True,
            skip_device_barrier=True,
        ),
        out_type=jax.ShapeDtypeStruct((_BATCH, _EMB), jnp.float32),
        scratch_types=[
            pltpu.VMEM((_BPW,), jnp.int32),
            pltpu.VMEM((_CH, _CB), jnp.int32),
            pltpu.VMEM_SHARED((_NROWS, _EMB), jnp.float32),
            pltpu.VMEM((_BPW, _EMB), jnp.float32),
            [pltpu.SemaphoreType.DMA] * _CH,
            pltpu.SemaphoreType.DMA,
            pltpu.SemaphoreType.DMA,
        ],
    )
    return run(values, emb_table)


# speedup vs baseline: 1.0428x; 1.0428x over previous
import jax
import jax.numpy as jnp
from jax import lax
from jax.experimental import pallas as pl
from jax.experimental.pallas import tpu as pltpu
from jax.experimental.pallas import tpu_sc as plsc

_MAX_VAL = 10
_EMB = 128
_BATCH = 16384
_NROWS = 2 * _MAX_VAL + 1

_NC = 2
_NS = 16
_NW = _NC * _NS
_BPW = _BATCH // _NW
_CH = 4
_CB = _BPW // _CH
_L = 16


def _body(values_hbm, table_hbm, out_hbm, vals_v, idx_v, table_sh, rows_v,
          gsems, wsem):
    sid = lax.axis_index("s")
    wid = sid * _NC + lax.axis_index("c")
    base = wid * _BPW
    pltpu.sync_copy(values_hbm.at[pl.ds(base, _BPW)], vals_v)

    for t, (lo, n) in enumerate([(0, 8), (8, 8), (16, _NROWS - 16)]):

        @pl.when(sid == t)
        def _stage(lo=lo, n=n):
            pltpu.sync_copy(
                table_hbm.at[pl.ds(lo, n)], table_sh.at[pl.ds(lo, n)]
            )

    for j in range(_CH):
        for k in range(_CB // _L):
            idx_v[j, pl.ds(k * _L, _L)] = (
                vals_v[pl.ds(j * _CB + k * _L, _L)] + _MAX_VAL
            )
    plsc.subcore_barrier()
    gathers = [
        pltpu.async_copy(
            table_sh.at[idx_v.at[j]], rows_v.at[pl.ds(j * _CB, _CB)], gsems[j]
        )
        for j in range(_CH)
    ]
    writes = []
    for j in range(_CH):
        gathers[j].wait()
        writes.append(
            pltpu.async_copy(
                rows_v.at[pl.ds(j * _CB, _CB)],
                out_hbm.at[pl.ds(base + j * _CB, _CB)],
                wsem,
            )
        )
    for w in writes:
        w.wait()


@jax.jit
def kernel(values, emb_table):
    run = pl.kernel(
        _body,
        mesh=plsc.VectorSubcoreMesh(core_axis_name="c", subcore_axis_name="s"),
        compiler_params=pltpu.CompilerParams(
            needs_layout_passes=False,
            disable_bounds_checks=True,
            disable_semaphore_checks=True,
            skip_device_barrier=True,
        ),
        out_type=jax.ShapeDtypeStruct((_BATCH, _EMB), jnp.float32),
        scratch_types=[
            pltpu.VMEM((_BPW,), jnp.int32),
            pltpu.VMEM((_CH, _CB), jnp.int32),
            pltpu.VMEM_SHARED((_NROWS, _EMB), jnp.float32),
            pltpu.VMEM((_BPW, _EMB), jnp.float32),
            [pltpu.SemaphoreType.DMA] * _CH,
            pltpu.SemaphoreType.DMA,
        ],
    )
    return run(values, emb_table)


# speedup vs baseline: 1.0475x; 1.0044x over previous
import jax
import jax.numpy as jnp
from jax import lax
from jax.experimental import pallas as pl
from jax.experimental.pallas import tpu as pltpu
from jax.experimental.pallas import tpu_sc as plsc

_MAX_VAL = 10
_EMB = 128
_BATCH = 16384
_NROWS = 2 * _MAX_VAL + 1

_NC = 2
_NS = 16
_NW = _NC * _NS
_BPW = _BATCH // _NW
_CH = 8
_CB = _BPW // _CH
_L = 16


def _body(values_hbm, table_hbm, out_hbm, vals_v, idx_v, table_sh, rows_v,
          gsems, wsem):
    sid = lax.axis_index("s")
    wid = sid * _NC + lax.axis_index("c")
    base = wid * _BPW
    pltpu.sync_copy(values_hbm.at[pl.ds(base, _BPW)], vals_v)

    for t, (lo, n) in enumerate([(0, 8), (8, 8), (16, _NROWS - 16)]):

        @pl.when(sid == t)
        def _stage(lo=lo, n=n):
            pltpu.sync_copy(
                table_hbm.at[pl.ds(lo, n)], table_sh.at[pl.ds(lo, n)]
            )

    for j in range(_CH):
        for k in range(_CB // _L):
            idx_v[j, pl.ds(k * _L, _L)] = (
                vals_v[pl.ds(j * _CB + k * _L, _L)] + _MAX_VAL
            )
    plsc.subcore_barrier()
    gathers = [
        pltpu.async_copy(
            table_sh.at[idx_v.at[j]], rows_v.at[pl.ds(j * _CB, _CB)], gsems[j]
        )
        for j in range(_CH)
    ]
    writes = []
    for j in range(_CH):
        gathers[j].wait()
        writes.append(
            pltpu.async_copy(
                rows_v.at[pl.ds(j * _CB, _CB)],
                out_hbm.at[pl.ds(base + j * _CB, _CB)],
                wsem,
            )
        )
    for w in writes:
        w.wait()


@jax.jit
def kernel(values, emb_table):
    run = pl.kernel(
        _body,
        mesh=plsc.VectorSubcoreMesh(core_axis_name="c", subcore_axis_name="s"),
        compiler_params=pltpu.CompilerParams(
            needs_layout_passes=False,
            disable_bounds_checks=True,
            disable_semaphore_checks=True,
            skip_device_barrier=True,
        ),
        out_type=jax.ShapeDtypeStruct((_BATCH, _EMB), jnp.float32),
        scratch_types=[
            pltpu.VMEM((_BPW,), jnp.int32),
            pltpu.VMEM((_CH, _CB), jnp.int32),
            pltpu.VMEM_SHARED((_NROWS, _EMB), jnp.float32),
            pltpu.VMEM((_BPW, _EMB), jnp.float32),
            [pltpu.SemaphoreType.DMA] * _CH,
            pltpu.SemaphoreType.DMA,
        ],
    )
    return run(values, emb_table)


# speedup vs baseline: 1.0525x; 1.0048x over previous
import jax
import jax.numpy as jnp
from jax import lax
from jax.experimental import pallas as pl
from jax.experimental.pallas import tpu as pltpu
from jax.experimental.pallas import tpu_sc as plsc

_MAX_VAL = 10
_EMB = 128
_BATCH = 16384
_NROWS = 2 * _MAX_VAL + 1

_NC = 2
_NS = 16
_NW = _NC * _NS
_BPW = _BATCH // _NW
_CH = 8
_CB = _BPW // _CH
_L = 16


def _body(values_hbm, table_hbm, out_hbm, vals_v, idx_v, table_sh, rows_v,
          gsems, wsem):
    sid = lax.axis_index("s")
    wid = sid * _NC + lax.axis_index("c")
    base = wid * _BPW
    pltpu.sync_copy(values_hbm.at[pl.ds(base, _BPW)], vals_v)

    for t, (lo, n) in enumerate([(0, 8), (8, 8), (16, _NROWS - 16)]):

        @pl.when(sid == t)
        def _stage(lo=lo, n=n):
            pltpu.sync_copy(
                table_hbm.at[pl.ds(lo, n)], table_sh.at[pl.ds(lo, n)]
            )

    for j in range(_CH):
        for k in range(_CB // _L):
            idx_v[j, pl.ds(k * _L, _L)] = (
                vals_v[pl.ds(j * _CB + k * _L, _L)] + _MAX_VAL
            )
    plsc.subcore_barrier()
    gathers = [
        pltpu.async_copy(
            table_sh.at[idx_v.at[j]], rows_v.at[pl.ds(j * _CB, _CB)], gsems[j]
        )
        for j in range(_CH)
    ]
    writes = []
    for j in range(_CH):
        gathers[j].wait()
        writes.append(
            pltpu.async_copy(
                rows_v.at[pl.ds(j * _CB, _CB)],
                out_hbm.at[pl.ds(base + j * _CB, _CB)],
                wsem,
            )
        )
    for w in writes:
        w.wait()


@jax.jit
def kernel(values, emb_table):
    run = pl.kernel(
        _body,
        mesh=plsc.VectorSubcoreMesh(core_axis_name="c", subcore_axis_name="s"),
        compiler_params=pltpu.CompilerParams(needs_layout_passes=False),
        out_type=jax.ShapeDtypeStruct((_BATCH, _EMB), jnp.float32),
        scratch_types=[
            pltpu.VMEM((_BPW,), jnp.int32),
            pltpu.VMEM((_CH, _CB), jnp.int32),
            pltpu.VMEM_SHARED((_NROWS, _EMB), jnp.float32),
            pltpu.VMEM((_BPW, _EMB), jnp.float32),
            [pltpu.SemaphoreType.DMA] * _CH,
            pltpu.SemaphoreType.DMA,
        ],
    )
    return run(values, emb_table)
